# baseline (speedup 1.0000x reference)
import functools

import jax
import jax.numpy as jnp
from jax import lax
from jax.experimental import pallas as pl
from jax.experimental.pallas import tpu as pltpu
from jax.experimental.pallas import tpu_sc as plsc

_BIG = 1e30


def _knn_body(ps_ref, pt_ref, nc_ref, wn_ref, idx_ref):
    ps = ps_ref[...]
    pt = pt_ref[0:3, :]
    ncp = pt_ref.shape[1]
    ns = jnp.sum(ps * ps, axis=1, keepdims=True)
    dot = jax.lax.dot_general(ps, pt, (((1,), (0,)), ((), ())),
                              preferred_element_type=jnp.float32)
    d2 = jnp.maximum(ns + nc_ref[...] - 2.0 * dot, 0.0)
    iota = lax.broadcasted_iota(jnp.int32, d2.shape, 1)
    m1 = jnp.min(d2, axis=1, keepdims=True)
    i1 = jnp.min(jnp.where(d2 <= m1, iota, ncp), axis=1, keepdims=True)
    mask1 = iota != i1
    m2 = jnp.min(jnp.where(mask1, d2, _BIG), axis=1, keepdims=True)
    i2 = jnp.min(jnp.where((d2 <= m2) & mask1, iota, ncp),
                 axis=1, keepdims=True)
    mask2 = mask1 & (iota != i2)
    m3 = jnp.min(jnp.where(mask2, d2, _BIG), axis=1, keepdims=True)
    i3 = jnp.min(jnp.where((d2 <= m3) & mask2, iota, ncp),
                 axis=1, keepdims=True)
    w1 = 1.0 / jnp.maximum(m1, 1e-16)
    w2 = 1.0 / jnp.maximum(m2, 1e-16)
    w3 = 1.0 / jnp.maximum(m3, 1e-16)
    den = w1 + w2 + w3
    z = jnp.zeros_like(w1)
    wn_ref[...] = jnp.concatenate(
        [w1 / den, w2 / den, w3 / den, z, z, z, z, z], axis=1)
    zi = jnp.zeros_like(i1)
    cap = jnp.int32(4999)
    idx_ref[...] = jnp.concatenate(
        [jnp.minimum(i1, cap), jnp.minimum(i2, cap), jnp.minimum(i3, cap),
         zi, zi, zi, zi, zi], axis=1)


def _mlp_body(h3_ref, wn_ref, xs_ref, w_ref, b_ref, out_ref):
    h = None
    for j in range(3):
        hj = h3_ref[j] * wn_ref[:, j:j + 1]
        h = hj if h is None else h + hj
    w1 = w_ref[0:128, :]
    w2 = w_ref[128:192, :]
    acc = jax.lax.dot_general(h, w1, (((1,), (0,)), ((), ())),
                              preferred_element_type=jnp.float32,
                              precision=jax.lax.Precision.HIGHEST)
    acc += jax.lax.dot_general(xs_ref[...], w2, (((1,), (0,)), ((), ())),
                               preferred_element_type=jnp.float32,
                               precision=jax.lax.Precision.HIGHEST)
    out_ref[...] = jnp.maximum(acc + b_ref[...], 0.0)


def _gather_sc(x, idx3, nf_pad):
    dx = x.shape[1]
    NW = 32
    BPW = nf_pad // NW
    CH = 128
    nchunk = BPW // CH
    mesh = plsc.VectorSubcoreMesh(core_axis_name="c", subcore_axis_name="s")

    @functools.partial(
        pl.kernel, mesh=mesh,
        out_type=jax.ShapeDtypeStruct((3, nf_pad, dx), jnp.float32),
        scratch_types=[
            pltpu.VMEM((BPW,), jnp.int32),
            pltpu.VMEM((BPW,), jnp.int32),
            pltpu.VMEM((BPW,), jnp.int32),
            pltpu.VMEM((CH, dx), jnp.float32),
            pltpu.SemaphoreType.DMA,
        ],
    )
    def k(x_hbm, idx_hbm, out_hbm, idx_v0, idx_v1, idx_v2, rows_v, sem):
        wid = lax.axis_index("s") * 2 + lax.axis_index("c")
        base = wid * BPW
        idx_vs = [idx_v0, idx_v1, idx_v2]
        for j in range(3):
            pltpu.sync_copy(idx_hbm.at[pl.ds(j * nf_pad + base, BPW)],
                            idx_vs[j])
        for j in range(3):
            for c in range(nchunk):
                pltpu.async_copy(
                    x_hbm.at[idx_vs[j].at[pl.ds(c * CH, CH)]], rows_v,
                    sem).wait()
                pltpu.sync_copy(
                    rows_v, out_hbm.at[j, pl.ds(base + c * CH, CH), :])

    return k(x, idx3)


@functools.partial(jax.jit, static_argnums=())
def kernel(x, pos, batch, x_skip, pos_skip, batch_skip, W, b):
    Nc, dx = x.shape
    Nf, dskip = x_skip.shape
    dout = W.shape[1]
    NCP = 5120
    F = 800
    NFP = 20480

    pos_t = jnp.zeros((8, NCP), dtype=jnp.float32).at[:3, :Nc].set(pos.T)
    nc_row = jnp.full((1, NCP), 1e10, dtype=jnp.float32).at[0, :Nc].set(
        jnp.sum(pos * pos, axis=1))

    grid = Nf // F
    wn, idxc = pl.pallas_call(
        _knn_body,
        grid=(grid,),
        in_specs=[
            pl.BlockSpec((F, 3), lambda i: (i, 0)),
            pl.BlockSpec((8, NCP), lambda i: (0, 0)),
            pl.BlockSpec((1, NCP), lambda i: (0, 0)),
        ],
        out_specs=[
            pl.BlockSpec((F, 8), lambda i: (i, 0)),
            pl.BlockSpec((F, 8), lambda i: (i, 0)),
        ],
        out_shape=[
            jax.ShapeDtypeStruct((Nf, 8), jnp.float32),
            jax.ShapeDtypeStruct((Nf, 8), jnp.int32),
        ],
    )(pos_skip, pos_t, nc_row)

    idx3 = jnp.zeros((3, NFP), dtype=jnp.int32).at[:, :Nf].set(
        idxc[:, :3].T)
    h3 = _gather_sc(x, idx3.reshape(3 * NFP), NFP)

    F2 = 2000
    out = pl.pallas_call(
        _mlp_body,
        grid=(Nf // F2,),
        in_specs=[
            pl.BlockSpec((3, F2, dx), lambda i: (0, i, 0)),
            pl.BlockSpec((F2, 8), lambda i: (i, 0)),
            pl.BlockSpec((F2, dskip), lambda i: (i, 0)),
            pl.BlockSpec((dx + dskip, dout), lambda i: (0, 0)),
            pl.BlockSpec((1, dout), lambda i: (0, 0)),
        ],
        out_specs=pl.BlockSpec((F2, dout), lambda i: (i, 0)),
        out_shape=jax.ShapeDtypeStruct((Nf, dout), jnp.float32),
    )(h3, wn, x_skip, W, b.reshape(1, dout))
    return (out, pos_skip, batch_skip)


# speedup vs baseline: 1.0154x; 1.0154x over previous
import functools

import jax
import jax.numpy as jnp
from jax import lax
from jax.experimental import pallas as pl
from jax.experimental.pallas import tpu as pltpu
from jax.experimental.pallas import tpu_sc as plsc

_BIG = 1e30


def _knn_body(ps_ref, pt_ref, nc_ref, wn_ref, idx_ref):
    ps = ps_ref[...]
    pt = pt_ref[0:3, :]
    ncp = pt_ref.shape[1]
    ns = jnp.sum(ps * ps, axis=1, keepdims=True)
    dot = jax.lax.dot_general(ps, pt, (((1,), (0,)), ((), ())),
                              preferred_element_type=jnp.float32)
    d2 = jnp.maximum(ns + nc_ref[...] - 2.0 * dot, 0.0)
    iota = lax.broadcasted_iota(jnp.int32, d2.shape, 1)
    m1 = jnp.min(d2, axis=1, keepdims=True)
    i1 = jnp.min(jnp.where(d2 <= m1, iota, ncp), axis=1, keepdims=True)
    mask1 = iota != i1
    m2 = jnp.min(jnp.where(mask1, d2, _BIG), axis=1, keepdims=True)
    i2 = jnp.min(jnp.where((d2 <= m2) & mask1, iota, ncp),
                 axis=1, keepdims=True)
    mask2 = mask1 & (iota != i2)
    m3 = jnp.min(jnp.where(mask2, d2, _BIG), axis=1, keepdims=True)
    i3 = jnp.min(jnp.where((d2 <= m3) & mask2, iota, ncp),
                 axis=1, keepdims=True)
    w1 = 1.0 / jnp.maximum(m1, 1e-16)
    w2 = 1.0 / jnp.maximum(m2, 1e-16)
    w3 = 1.0 / jnp.maximum(m3, 1e-16)
    den = w1 + w2 + w3
    z = jnp.zeros_like(w1)
    wn_ref[...] = jnp.concatenate(
        [w1 / den, w2 / den, w3 / den, z, z, z, z, z], axis=1)
    zi = jnp.zeros_like(i1)
    cap = jnp.int32(4999)
    idx_ref[...] = jnp.concatenate(
        [jnp.minimum(i1, cap), jnp.minimum(i2, cap), jnp.minimum(i3, cap),
         zi, zi, zi, zi, zi], axis=1)


def _mlp_body(h3_ref, wn_ref, xs_ref, w_ref, b_ref, out_ref):
    h = None
    for j in range(3):
        hj = h3_ref[j] * wn_ref[:, j:j + 1]
        h = hj if h is None else h + hj
    w1 = w_ref[0:128, :]
    w2 = w_ref[128:192, :]
    acc = jax.lax.dot_general(h, w1, (((1,), (0,)), ((), ())),
                              preferred_element_type=jnp.float32,
                              precision=jax.lax.Precision.HIGHEST)
    acc += jax.lax.dot_general(xs_ref[...], w2, (((1,), (0,)), ((), ())),
                               preferred_element_type=jnp.float32,
                               precision=jax.lax.Precision.HIGHEST)
    out_ref[...] = jnp.maximum(acc + b_ref[...], 0.0)


def _gather_sc(x, idx3, nf_pad):
    dx = x.shape[1]
    NW = 32
    BPW = nf_pad // NW
    CH = 128
    nchunk = BPW // CH
    mesh = plsc.VectorSubcoreMesh(core_axis_name="c", subcore_axis_name="s")

    @functools.partial(
        pl.kernel, mesh=mesh,
        out_type=jax.ShapeDtypeStruct((3, nf_pad, dx), jnp.float32),
        scratch_types=[
            pltpu.VMEM((BPW,), jnp.int32),
            pltpu.VMEM((BPW,), jnp.int32),
            pltpu.VMEM((BPW,), jnp.int32),
            pltpu.VMEM((CH, dx), jnp.float32),
            pltpu.VMEM((CH, dx), jnp.float32),
            pltpu.SemaphoreType.DMA,
            pltpu.SemaphoreType.DMA,
        ],
    )
    def k(x_hbm, idx_hbm, out_hbm, idx_v0, idx_v1, idx_v2,
          rows_a, rows_b, sem_a, sem_b):
        wid = lax.axis_index("s") * 2 + lax.axis_index("c")
        base = wid * BPW
        idx_vs = [idx_v0, idx_v1, idx_v2]
        for j in range(3):
            pltpu.sync_copy(idx_hbm.at[pl.ds(j * nf_pad + base, BPW)],
                            idx_vs[j])
        steps = [(j, c) for j in range(3) for c in range(nchunk)]
        bufs = (rows_a, rows_b)
        sems = (sem_a, sem_b)

        def start(s):
            j, c = steps[s]
            return pltpu.async_copy(
                x_hbm.at[idx_vs[j].at[pl.ds(c * CH, CH)]],
                bufs[s % 2], sems[s % 2])

        cp = start(0)
        for s in range(len(steps)):
            cp.wait()
            if s + 1 < len(steps):
                cp = start(s + 1)
            j, c = steps[s]
            pltpu.sync_copy(bufs[s % 2],
                            out_hbm.at[j, pl.ds(base + c * CH, CH), :])

    return k(x, idx3)


@functools.partial(jax.jit, static_argnums=())
def kernel(x, pos, batch, x_skip, pos_skip, batch_skip, W, b):
    Nc, dx = x.shape
    Nf, dskip = x_skip.shape
    dout = W.shape[1]
    NCP = 5120
    F = 800
    NFP = 20480

    pos_t = jnp.zeros((8, NCP), dtype=jnp.float32).at[:3, :Nc].set(pos.T)
    nc_row = jnp.full((1, NCP), 1e10, dtype=jnp.float32).at[0, :Nc].set(
        jnp.sum(pos * pos, axis=1))

    grid = Nf // F
    wn, idxc = pl.pallas_call(
        _knn_body,
        grid=(grid,),
        in_specs=[
            pl.BlockSpec((F, 3), lambda i: (i, 0)),
            pl.BlockSpec((8, NCP), lambda i: (0, 0)),
            pl.BlockSpec((1, NCP), lambda i: (0, 0)),
        ],
        out_specs=[
            pl.BlockSpec((F, 8), lambda i: (i, 0)),
            pl.BlockSpec((F, 8), lambda i: (i, 0)),
        ],
        out_shape=[
            jax.ShapeDtypeStruct((Nf, 8), jnp.float32),
            jax.ShapeDtypeStruct((Nf, 8), jnp.int32),
        ],
    )(pos_skip, pos_t, nc_row)

    idx3 = jnp.zeros((3, NFP), dtype=jnp.int32).at[:, :Nf].set(
        idxc[:, :3].T)
    h3 = _gather_sc(x, idx3.reshape(3 * NFP), NFP)

    F2 = 2000
    out = pl.pallas_call(
        _mlp_body,
        grid=(Nf // F2,),
        in_specs=[
            pl.BlockSpec((3, F2, dx), lambda i: (0, i, 0)),
            pl.BlockSpec((F2, 8), lambda i: (i, 0)),
            pl.BlockSpec((F2, dskip), lambda i: (i, 0)),
            pl.BlockSpec((dx + dskip, dout), lambda i: (0, 0)),
            pl.BlockSpec((1, dout), lambda i: (0, 0)),
        ],
        out_specs=pl.BlockSpec((F2, dout), lambda i: (i, 0)),
        out_shape=jax.ShapeDtypeStruct((Nf, dout), jnp.float32),
    )(h3, wn, x_skip, W, b.reshape(1, dout))
    return (out, pos_skip, batch_skip)


# speedup vs baseline: 1.0185x; 1.0031x over previous
import functools

import jax
import jax.numpy as jnp
from jax import lax
from jax.experimental import pallas as pl
from jax.experimental.pallas import tpu as pltpu
from jax.experimental.pallas import tpu_sc as plsc

_BIG = 1e30


def _knn_body(ps_ref, pt_ref, nc_ref, wn_ref, idx_ref):
    ps = ps_ref[...]
    pt = pt_ref[0:3, :]
    ncp = pt_ref.shape[1]
    ns = jnp.sum(ps * ps, axis=1, keepdims=True)
    dot = jax.lax.dot_general(ps, pt, (((1,), (0,)), ((), ())),
                              preferred_element_type=jnp.float32)
    d2 = nc_ref[...] - 2.0 * dot
    iota = lax.broadcasted_iota(jnp.int32, d2.shape, 1)
    m1 = jnp.min(d2, axis=1, keepdims=True)
    i1 = jnp.min(jnp.where(d2 <= m1, iota, ncp), axis=1, keepdims=True)
    mask1 = iota != i1
    m2 = jnp.min(jnp.where(mask1, d2, _BIG), axis=1, keepdims=True)
    i2 = jnp.min(jnp.where((d2 <= m2) & mask1, iota, ncp),
                 axis=1, keepdims=True)
    mask2 = mask1 & (iota != i2)
    m3 = jnp.min(jnp.where(mask2, d2, _BIG), axis=1, keepdims=True)
    i3 = jnp.min(jnp.where((d2 <= m3) & mask2, iota, ncp),
                 axis=1, keepdims=True)
    w1 = 1.0 / jnp.maximum(jnp.maximum(m1 + ns, 0.0), 1e-16)
    w2 = 1.0 / jnp.maximum(jnp.maximum(m2 + ns, 0.0), 1e-16)
    w3 = 1.0 / jnp.maximum(jnp.maximum(m3 + ns, 0.0), 1e-16)
    den = w1 + w2 + w3
    z = jnp.zeros_like(w1)
    wn_ref[...] = jnp.concatenate(
        [w1 / den, w2 / den, w3 / den, z, z, z, z, z], axis=1)
    zi = jnp.zeros_like(i1)
    cap = jnp.int32(4999)
    idx_ref[...] = jnp.concatenate(
        [jnp.minimum(i1, cap), jnp.minimum(i2, cap), jnp.minimum(i3, cap),
         zi, zi, zi, zi, zi], axis=1)


def _mlp_body(h3_ref, wn_ref, xs_ref, w_ref, b_ref, out_ref):
    h = None
    for j in range(3):
        hj = h3_ref[j] * wn_ref[:, j:j + 1]
        h = hj if h is None else h + hj
    w1 = w_ref[0:128, :]
    w2 = w_ref[128:192, :]
    acc = jax.lax.dot_general(h, w1, (((1,), (0,)), ((), ())),
                              preferred_element_type=jnp.float32,
                              precision=jax.lax.Precision.HIGHEST)
    acc += jax.lax.dot_general(xs_ref[...], w2, (((1,), (0,)), ((), ())),
                               preferred_element_type=jnp.float32,
                               precision=jax.lax.Precision.HIGHEST)
    out_ref[...] = jnp.maximum(acc + b_ref[...], 0.0)


def _gather_sc(x, idx3, nf_pad):
    dx = x.shape[1]
    NW = 32
    BPW = nf_pad // NW
    CH = 128
    nchunk = BPW // CH
    mesh = plsc.VectorSubcoreMesh(core_axis_name="c", subcore_axis_name="s")

    @functools.partial(
        pl.kernel, mesh=mesh,
        out_type=jax.ShapeDtypeStruct((3, nf_pad, dx), jnp.float32),
        scratch_types=[
            pltpu.VMEM((BPW,), jnp.int32),
            pltpu.VMEM((BPW,), jnp.int32),
            pltpu.VMEM((BPW,), jnp.int32),
            pltpu.VMEM((CH, dx), jnp.float32),
            pltpu.VMEM((CH, dx), jnp.float32),
            pltpu.SemaphoreType.DMA,
            pltpu.SemaphoreType.DMA,
        ],
    )
    def k(x_hbm, idx_hbm, out_hbm, idx_v0, idx_v1, idx_v2,
          rows_a, rows_b, sem_a, sem_b):
        wid = lax.axis_index("s") * 2 + lax.axis_index("c")
        base = wid * BPW
        idx_vs = [idx_v0, idx_v1, idx_v2]
        for j in range(3):
            pltpu.sync_copy(idx_hbm.at[pl.ds(j * nf_pad + base, BPW)],
                            idx_vs[j])
        steps = [(j, c) for j in range(3) for c in range(nchunk)]
        bufs = (rows_a, rows_b)
        sems = (sem_a, sem_b)

        def start(s):
            j, c = steps[s]
            return pltpu.async_copy(
                x_hbm.at[idx_vs[j].at[pl.ds(c * CH, CH)]],
                bufs[s % 2], sems[s % 2])

        cp = start(0)
        for s in range(len(steps)):
            cp.wait()
            if s + 1 < len(steps):
                cp = start(s + 1)
            j, c = steps[s]
            pltpu.sync_copy(bufs[s % 2],
                            out_hbm.at[j, pl.ds(base + c * CH, CH), :])

    return k(x, idx3)


@functools.partial(jax.jit, static_argnums=())
def kernel(x, pos, batch, x_skip, pos_skip, batch_skip, W, b):
    Nc, dx = x.shape
    Nf, dskip = x_skip.shape
    dout = W.shape[1]
    NCP = 5120
    F = 800
    NFP = 20480

    pos_t = jnp.zeros((8, NCP), dtype=jnp.float32).at[:3, :Nc].set(pos.T)
    nc_row = jnp.full((1, NCP), 1e10, dtype=jnp.float32).at[0, :Nc].set(
        jnp.sum(pos * pos, axis=1))

    grid = Nf // F
    wn, idxc = pl.pallas_call(
        _knn_body,
        grid=(grid,),
        in_specs=[
            pl.BlockSpec((F, 3), lambda i: (i, 0)),
            pl.BlockSpec((8, NCP), lambda i: (0, 0)),
            pl.BlockSpec((1, NCP), lambda i: (0, 0)),
        ],
        out_specs=[
            pl.BlockSpec((F, 8), lambda i: (i, 0)),
            pl.BlockSpec((F, 8), lambda i: (i, 0)),
        ],
        out_shape=[
            jax.ShapeDtypeStruct((Nf, 8), jnp.float32),
            jax.ShapeDtypeStruct((Nf, 8), jnp.int32),
        ],
    )(pos_skip, pos_t, nc_row)

    idx3 = jnp.zeros((3, NFP), dtype=jnp.int32).at[:, :Nf].set(
        idxc[:, :3].T)
    h3 = _gather_sc(x, idx3.reshape(3 * NFP), NFP)

    F2 = 2000
    out = pl.pallas_call(
        _mlp_body,
        grid=(Nf // F2,),
        in_specs=[
            pl.BlockSpec((3, F2, dx), lambda i: (0, i, 0)),
            pl.BlockSpec((F2, 8), lambda i: (i, 0)),
            pl.BlockSpec((F2, dskip), lambda i: (i, 0)),
            pl.BlockSpec((dx + dskip, dout), lambda i: (0, 0)),
            pl.BlockSpec((1, dout), lambda i: (0, 0)),
        ],
        out_specs=pl.BlockSpec((F2, dout), lambda i: (i, 0)),
        out_shape=jax.ShapeDtypeStruct((Nf, dout), jnp.float32),
    )(h3, wn, x_skip, W, b.reshape(1, dout))
    return (out, pos_skip, batch_skip)
